# Initial kernel scaffold; baseline (speedup 1.0000x reference)
import jax
import jax.numpy as jnp
from jax.experimental import pallas as pl


def kernel(h1, h2):
    raise NotImplementedError("write your pallas kernel here")


# speedup vs baseline: 1.3042x; 1.3042x over previous
import functools

import jax
import jax.numpy as jnp
from jax.experimental import pallas as pl
from jax.experimental.pallas import tpu as pltpu

_TAU = 0.5
_BI = 512
_BJ = 1024


def _body(x1i_ref, x2i_ref, x1j_ref, x2j_ref, s1_ref, s2_ref, diag_ref):
    i = pl.program_id(0)
    j = pl.program_id(1)

    x1i = x1i_ref[...]
    x2i = x2i_ref[...]
    x1j = x1j_ref[...]
    x2j = x2j_ref[...]

    dn = (((1,), (1,)), ((), ()))
    a = jax.lax.dot_general(x1i, x1j, dn, preferred_element_type=jnp.float32)
    c = jax.lax.dot_general(x1i, x2j, dn, preferred_element_type=jnp.float32)
    b = jax.lax.dot_general(x2i, x2j, dn, preferred_element_type=jnp.float32)
    d = jax.lax.dot_general(x2i, x1j, dn, preferred_element_type=jnp.float32)

    inv_tau = jnp.float32(1.0 / _TAU)
    s1_blk = jnp.sum(jnp.exp(a * inv_tau) + jnp.exp(c * inv_tau), axis=1)
    s2_blk = jnp.sum(jnp.exp(b * inv_tau) + jnp.exp(d * inv_tau), axis=1)

    @pl.when(j == 0)
    def _init():
        s1_ref[...] = jnp.zeros_like(s1_ref)
        s2_ref[...] = jnp.zeros_like(s2_ref)

    s1_ref[0, 0, :] += s1_blk
    s2_ref[0, 0, :] += s2_blk

    @pl.when(i * _BI == j * _BJ)
    def _diag():
        diag_ref[0, 0, :] = jnp.sum(
            x1i.astype(jnp.float32) * x2i.astype(jnp.float32), axis=1
        )


@functools.partial(jax.jit, static_argnums=(2, 3))
def _rowsums(z1, z2, np_, d):
    ti = np_ // _BI
    tj = np_ // _BJ
    grid = (ti, tj)
    in_spec_i = pl.BlockSpec((_BI, d), lambda i, j: (i, 0))
    in_spec_j = pl.BlockSpec((_BJ, d), lambda i, j: (j, 0))
    out_spec = pl.BlockSpec((1, 1, _BI), lambda i, j: (i, 0, 0))
    out_shape = jax.ShapeDtypeStruct((ti, 1, _BI), jnp.float32)
    s1, s2, diag = pl.pallas_call(
        _body,
        grid=grid,
        in_specs=[in_spec_i, in_spec_i, in_spec_j, in_spec_j],
        out_specs=[out_spec, out_spec, out_spec],
        out_shape=[out_shape, out_shape, out_shape],
        compiler_params=pltpu.CompilerParams(
            dimension_semantics=("parallel", "arbitrary"),
        ),
    )(z1, z2, z1, z2)
    return s1.reshape(-1), s2.reshape(-1), diag.reshape(-1)


def kernel(h1, h2):
    n, d = h1.shape
    z1 = h1 / jnp.linalg.norm(h1, axis=1, keepdims=True)
    z2 = h2 / jnp.linalg.norm(h2, axis=1, keepdims=True)

    blk = max(_BI, _BJ)
    np_ = ((n + blk - 1) // blk) * blk
    pad = np_ - n
    z1p = jnp.pad(z1, ((0, pad), (0, 0))).astype(jnp.bfloat16)
    z2p = jnp.pad(z2, ((0, pad), (0, 0))).astype(jnp.bfloat16)

    s1, s2, diag = _rowsums(z1p, z2p, np_, d)
    s1 = s1[:n]
    s2 = s2[:n]
    diag = diag[:n]

    inv_tau = jnp.float32(1.0 / _TAU)
    self_sim = jnp.exp(inv_tau)
    pad_ones = jnp.float32(2 * pad)
    denom1 = s1 - pad_ones - self_sim
    denom2 = s2 - pad_ones - self_sim
    log_pos = diag * inv_tau
    l1 = jnp.mean(jnp.log(denom1) - log_pos)
    l2 = jnp.mean(jnp.log(denom2) - log_pos)
    return (l1 + l2) * jnp.float32(0.5)


# speedup vs baseline: 2.0965x; 1.6076x over previous
import functools

import jax
import jax.numpy as jnp
from jax.experimental import pallas as pl
from jax.experimental.pallas import tpu as pltpu

_TAU = 0.5
_B = 1024


def _body(bi_ref, bj_ref, wi_ref, wj_ref, out_ref, acc_ref, *, num_pairs, t_blocks):
    t = pl.program_id(0)
    bi = bi_ref[t]
    bj = bj_ref[t]

    @pl.when(t == 0)
    def _init():
        acc_ref[...] = jnp.zeros_like(acc_ref)

    dn = (((1,), (1,)), ((), ()))
    s = jax.lax.dot_general(
        wi_ref[...], wj_ref[...], dn, preferred_element_type=jnp.float32
    )
    e = jnp.exp2(s)

    sub0 = jax.lax.broadcasted_iota(jnp.int32, (8, _B), 0) == 0
    row = jnp.sum(e, axis=1)
    rowm = jnp.where(sub0, row[None, :], jnp.float32(0.0))
    acc_ref[pl.ds(bi, 1)] += rowm[None, :, :]

    @pl.when(bj != bi)
    def _col():
        col = jnp.sum(e, axis=0)
        colm = jnp.where(sub0, col[None, :], jnp.float32(0.0))
        acc_ref[pl.ds(bj, 1)] += colm[None, :, :]

    @pl.when(t == num_pairs - 1)
    def _flush():
        out_ref[...] = acc_ref[...]


@functools.partial(jax.jit, static_argnums=(1,))
def _rowsums(w, m):
    t_blocks = m // _B
    pairs = [(i, j) for i in range(t_blocks) for j in range(i, t_blocks)]
    num_pairs = len(pairs)
    bi_arr = jnp.asarray([p[0] for p in pairs], dtype=jnp.int32)
    bj_arr = jnp.asarray([p[1] for p in pairs], dtype=jnp.int32)

    d = w.shape[1]
    grid_spec = pltpu.PrefetchScalarGridSpec(
        num_scalar_prefetch=2,
        grid=(num_pairs,),
        in_specs=[
            pl.BlockSpec((_B, d), lambda t, bi, bj: (bi[t], 0)),
            pl.BlockSpec((_B, d), lambda t, bi, bj: (bj[t], 0)),
        ],
        out_specs=pl.BlockSpec((t_blocks, 8, _B), lambda t, bi, bj: (0, 0, 0)),
        scratch_shapes=[pltpu.VMEM((t_blocks, 8, _B), jnp.float32)],
    )
    out = pl.pallas_call(
        functools.partial(_body, num_pairs=num_pairs, t_blocks=t_blocks),
        grid_spec=grid_spec,
        out_shape=jax.ShapeDtypeStruct((t_blocks, 8, _B), jnp.float32),
        compiler_params=pltpu.CompilerParams(
            dimension_semantics=("arbitrary",),
        ),
    )(bi_arr, bj_arr, w, w)
    return out[:, 0, :].reshape(m)


def kernel(h1, h2):
    n, d = h1.shape
    z1 = h1 / jnp.linalg.norm(h1, axis=1, keepdims=True)
    z2 = h2 / jnp.linalg.norm(h2, axis=1, keepdims=True)

    inv_tau = jnp.float32(1.0 / _TAU)
    c = float(1.0 / _TAU) * 1.4426950408889634
    sqrt_c = c ** 0.5

    np_ = ((n + _B - 1) // _B) * _B
    pad = np_ - n
    z1p = jnp.pad(z1 * sqrt_c, ((0, pad), (0, 0))).astype(jnp.bfloat16)
    z2p = jnp.pad(z2 * sqrt_c, ((0, pad), (0, 0))).astype(jnp.bfloat16)
    w = jnp.concatenate([z1p, z2p], axis=0)
    m = 2 * np_

    g = _rowsums(w, m)
    s1 = g[:n]
    s2 = g[np_ : np_ + n]

    self_sim = jnp.exp(inv_tau)
    pad_ones = jnp.float32(2 * pad)
    denom1 = s1 - pad_ones - self_sim
    denom2 = s2 - pad_ones - self_sim
    log_pos = jnp.sum(z1 * z2, axis=1) * inv_tau
    l1 = jnp.mean(jnp.log(denom1) - log_pos)
    l2 = jnp.mean(jnp.log(denom2) - log_pos)
    return (l1 + l2) * jnp.float32(0.5)


# speedup vs baseline: 2.5690x; 1.2254x over previous
import functools

import jax
import jax.numpy as jnp
from jax.experimental import pallas as pl
from jax.experimental.pallas import tpu as pltpu

_TAU = 0.5
_B = 2048


def _body(bi_ref, bj_ref, wi_ref, wj_ref, out_ref, acc_ref, *, num_pairs, t_blocks):
    t = pl.program_id(0)
    bi = bi_ref[t]
    bj = bj_ref[t]

    @pl.when(t == 0)
    def _init():
        acc_ref[...] = jnp.zeros_like(acc_ref)

    dn = (((1,), (1,)), ((), ()))
    s = jax.lax.dot_general(
        wi_ref[...], wj_ref[...], dn, preferred_element_type=jnp.float32
    )
    e = jnp.exp2(s)

    sub0 = jax.lax.broadcasted_iota(jnp.int32, (8, _B), 0) == 0
    row = jnp.sum(e, axis=1)
    rowm = jnp.where(sub0, row[None, :], jnp.float32(0.0))
    acc_ref[pl.ds(bi, 1)] += rowm[None, :, :]

    @pl.when(bj != bi)
    def _col():
        col = jnp.sum(e, axis=0)
        colm = jnp.where(sub0, col[None, :], jnp.float32(0.0))
        acc_ref[pl.ds(bj, 1)] += colm[None, :, :]

    @pl.when(t == num_pairs - 1)
    def _flush():
        out_ref[...] = acc_ref[...]


@functools.partial(jax.jit, static_argnums=(1,))
def _rowsums(w, m):
    t_blocks = m // _B
    pairs = [(i, j) for i in range(t_blocks) for j in range(i, t_blocks)]
    num_pairs = len(pairs)
    bi_arr = jnp.asarray([p[0] for p in pairs], dtype=jnp.int32)
    bj_arr = jnp.asarray([p[1] for p in pairs], dtype=jnp.int32)

    d = w.shape[1]
    grid_spec = pltpu.PrefetchScalarGridSpec(
        num_scalar_prefetch=2,
        grid=(num_pairs,),
        in_specs=[
            pl.BlockSpec((_B, d), lambda t, bi, bj: (bi[t], 0)),
            pl.BlockSpec((_B, d), lambda t, bi, bj: (bj[t], 0)),
        ],
        out_specs=pl.BlockSpec((t_blocks, 8, _B), lambda t, bi, bj: (0, 0, 0)),
        scratch_shapes=[pltpu.VMEM((t_blocks, 8, _B), jnp.float32)],
    )
    out = pl.pallas_call(
        functools.partial(_body, num_pairs=num_pairs, t_blocks=t_blocks),
        grid_spec=grid_spec,
        out_shape=jax.ShapeDtypeStruct((t_blocks, 8, _B), jnp.float32),
        compiler_params=pltpu.CompilerParams(
            dimension_semantics=("arbitrary",),
        ),
    )(bi_arr, bj_arr, w, w)
    return out[:, 0, :].reshape(m)


def kernel(h1, h2):
    n, d = h1.shape
    z1 = h1 / jnp.linalg.norm(h1, axis=1, keepdims=True)
    z2 = h2 / jnp.linalg.norm(h2, axis=1, keepdims=True)

    inv_tau = jnp.float32(1.0 / _TAU)
    c = float(1.0 / _TAU) * 1.4426950408889634
    sqrt_c = c ** 0.5

    np_ = ((n + _B - 1) // _B) * _B
    pad = np_ - n
    z1p = jnp.pad(z1 * sqrt_c, ((0, pad), (0, 0))).astype(jnp.bfloat16)
    z2p = jnp.pad(z2 * sqrt_c, ((0, pad), (0, 0))).astype(jnp.bfloat16)
    w = jnp.concatenate([z1p, z2p], axis=0)
    m = 2 * np_

    g = _rowsums(w, m)
    s1 = g[:n]
    s2 = g[np_ : np_ + n]

    self_sim = jnp.exp(inv_tau)
    pad_ones = jnp.float32(2 * pad)
    denom1 = s1 - pad_ones - self_sim
    denom2 = s2 - pad_ones - self_sim
    log_pos = jnp.sum(z1 * z2, axis=1) * inv_tau
    l1 = jnp.mean(jnp.log(denom1) - log_pos)
    l2 = jnp.mean(jnp.log(denom2) - log_pos)
    return (l1 + l2) * jnp.float32(0.5)


# speedup vs baseline: 2.8840x; 1.1226x over previous
import functools

import jax
import jax.numpy as jnp
from jax.experimental import pallas as pl
from jax.experimental.pallas import tpu as pltpu

_TAU = 0.5
_B = 2048
_CH = 512


def _body(bi_ref, bj_ref, wi_ref, wj_ref, outr_ref, outc_ref, *, num_pairs):
    t = pl.program_id(0)
    bi = bi_ref[t]
    bj = bj_ref[t]

    @pl.when(t == 0)
    def _init():
        outr_ref[...] = jnp.zeros_like(outr_ref)
        outc_ref[...] = jnp.zeros_like(outc_ref)

    dn = (((1,), (1,)), ((), ()))
    wj = wj_ref[...]

    col8 = jnp.zeros((8, _B), jnp.float32)
    for r in range(_B // _CH):
        wi_r = wi_ref[pl.ds(r * _CH, _CH), :]
        s_r = jax.lax.dot_general(wi_r, wj, dn, preferred_element_type=jnp.float32)
        e_r = jnp.exp2(s_r)
        rowp = e_r[:, 0:128]
        for g in range(1, _B // 128):
            rowp = rowp + e_r[:, g * 128 : (g + 1) * 128]
        outr_ref[pl.ds(bi, 1), pl.ds(r * _CH, _CH), :] += rowp[None]
        cpart = e_r[0:8, :]
        for k in range(1, _CH // 8):
            cpart = cpart + e_r[k * 8 : (k + 1) * 8, :]
        col8 = col8 + cpart

    @pl.when(bj != bi)
    def _col():
        outc_ref[pl.ds(bj, 1)] += col8[None, :, :]


@functools.partial(jax.jit, static_argnums=(1,))
def _rowsums(w, m):
    t_blocks = m // _B
    pairs = [(i, j) for i in range(t_blocks) for j in range(i, t_blocks)]
    num_pairs = len(pairs)
    bi_arr = jnp.asarray([p[0] for p in pairs], dtype=jnp.int32)
    bj_arr = jnp.asarray([p[1] for p in pairs], dtype=jnp.int32)

    d = w.shape[1]
    grid_spec = pltpu.PrefetchScalarGridSpec(
        num_scalar_prefetch=2,
        grid=(num_pairs,),
        in_specs=[
            pl.BlockSpec((_B, d), lambda t, bi, bj: (bi[t], 0)),
            pl.BlockSpec((_B, d), lambda t, bi, bj: (bj[t], 0)),
        ],
        out_specs=[
            pl.BlockSpec((t_blocks, _B, 128), lambda t, bi, bj: (0, 0, 0)),
            pl.BlockSpec((t_blocks, 8, _B), lambda t, bi, bj: (0, 0, 0)),
        ],
        scratch_shapes=[],
    )
    outr, outc = pl.pallas_call(
        functools.partial(_body, num_pairs=num_pairs),
        grid_spec=grid_spec,
        out_shape=[
            jax.ShapeDtypeStruct((t_blocks, _B, 128), jnp.float32),
            jax.ShapeDtypeStruct((t_blocks, 8, _B), jnp.float32),
        ],
        compiler_params=pltpu.CompilerParams(
            dimension_semantics=("arbitrary",),
        ),
    )(bi_arr, bj_arr, w, w)
    return jnp.sum(outr, axis=2).reshape(m) + jnp.sum(outc, axis=1).reshape(m)


def kernel(h1, h2):
    n, d = h1.shape
    z1 = h1 / jnp.linalg.norm(h1, axis=1, keepdims=True)
    z2 = h2 / jnp.linalg.norm(h2, axis=1, keepdims=True)

    inv_tau = jnp.float32(1.0 / _TAU)
    c = float(1.0 / _TAU) * 1.4426950408889634
    sqrt_c = c ** 0.5

    np_ = ((n + _B - 1) // _B) * _B
    pad = np_ - n
    z1p = jnp.pad(z1 * sqrt_c, ((0, pad), (0, 0))).astype(jnp.bfloat16)
    z2p = jnp.pad(z2 * sqrt_c, ((0, pad), (0, 0))).astype(jnp.bfloat16)
    w = jnp.concatenate([z1p, z2p], axis=0)
    m = 2 * np_

    g = _rowsums(w, m)
    s1 = g[:n]
    s2 = g[np_ : np_ + n]

    self_sim = jnp.exp(inv_tau)
    pad_ones = jnp.float32(2 * pad)
    denom1 = s1 - pad_ones - self_sim
    denom2 = s2 - pad_ones - self_sim
    log_pos = jnp.sum(z1 * z2, axis=1) * inv_tau
    l1 = jnp.mean(jnp.log(denom1) - log_pos)
    l2 = jnp.mean(jnp.log(denom2) - log_pos)
    return (l1 + l2) * jnp.float32(0.5)
